# baseline (speedup 1.0000x reference)
import functools
import math

import jax
import jax.numpy as jnp
from jax.experimental import pallas as pl

CLASS_TYPE_STATIC = 2
CLASS_TYPE_NUM = 7
DIM = 10
HIDDEN = 64
POS_DIM = 4 + CLASS_TYPE_NUM

_SQRT_HALF = 1.0 / math.sqrt(2.0)


def _fused_kernel(x_ref, w1_ref, b1_ref, w2_ref, b2_ref,
                  out_ref, mask_ref, pos_ref):
    x = x_ref[...]

    nonzero = jnp.sum((x != 0.0).astype(jnp.float32), axis=-1, keepdims=True)
    maskf = (nonzero == 0.0).astype(jnp.float32)
    mask_ref[...] = maskf

    tile = x.shape[0]
    col = jax.lax.broadcasted_iota(jnp.int32, (tile, CLASS_TYPE_NUM), 1)
    onehot = (col == CLASS_TYPE_STATIC).astype(jnp.float32)
    pos_ref[...] = jnp.concatenate([x[:, :4], onehot], axis=-1)

    h = jnp.dot(x, w1_ref[...], preferred_element_type=jnp.float32) + b1_ref[...]
    h = 0.5 * h * (1.0 + jax.lax.erf(h * _SQRT_HALF))
    proj = jnp.dot(h, w2_ref[...], preferred_element_type=jnp.float32) + b2_ref[...]
    out_ref[...] = proj * (1.0 - maskf)


@functools.partial(jax.jit, static_argnames=())
def kernel(x, W1, b1, W2, b2):
    B, P, D = x.shape
    N = B * P
    TILE = 8192
    xf = x.reshape(N, D)

    grid = (N // TILE,)
    out, maskf, pos = pl.pallas_call(
        _fused_kernel,
        grid=grid,
        in_specs=[
            pl.BlockSpec((TILE, D), lambda i: (i, 0)),
            pl.BlockSpec((D, HIDDEN), lambda i: (0, 0)),
            pl.BlockSpec((1, HIDDEN), lambda i: (0, 0)),
            pl.BlockSpec((HIDDEN, HIDDEN), lambda i: (0, 0)),
            pl.BlockSpec((1, HIDDEN), lambda i: (0, 0)),
        ],
        out_specs=[
            pl.BlockSpec((TILE, HIDDEN), lambda i: (i, 0)),
            pl.BlockSpec((TILE, 1), lambda i: (i, 0)),
            pl.BlockSpec((TILE, POS_DIM), lambda i: (i, 0)),
        ],
        out_shape=[
            jax.ShapeDtypeStruct((N, HIDDEN), jnp.float32),
            jax.ShapeDtypeStruct((N, 1), jnp.float32),
            jax.ShapeDtypeStruct((N, POS_DIM), jnp.float32),
        ],
    )(xf, W1, b1.reshape(1, HIDDEN), W2, b2.reshape(1, HIDDEN))

    return (out.reshape(B, P, HIDDEN),
            maskf.reshape(B, P).astype(jnp.bool_),
            pos.reshape(B, P, POS_DIM))
